# Initial kernel scaffold; baseline (speedup 1.0000x reference)
import functools

import jax
import jax.numpy as jnp
from jax import lax
from jax.experimental import pallas as pl
from jax.experimental.pallas import tpu as pltpu
from jax.experimental.pallas import tpu_sc as plsc

_INFO = plsc.get_sparse_core_info()
_NC, _NS = _INFO.num_cores, _INFO.num_subcores
_NW = _NC * _NS

_B = 4096
_SEQ = 200
_D = 32
_ROWS_PER_W = _B // _NW
_IPW = _ROWS_PER_W * _SEQ


def _body(ids_hbm, table_hbm, pos_hbm, out_hbm, idx_v, pos_v, rows_v, sem):
    wid = lax.axis_index("s") * _NC + lax.axis_index("c")
    base = wid * _IPW

    pltpu.sync_copy(ids_hbm.at[pl.ds(base, _IPW)], idx_v)
    pltpu.sync_copy(pos_hbm.at[pl.ds(0, _SEQ)], pos_v)

    def row_body(r, _):
        pltpu.async_copy(
            table_hbm.at[idx_v.at[pl.ds(r * _SEQ, _SEQ)]], rows_v, sem
        ).wait()

        def add_body(s, _):
            plsc.addupdate(rows_v.at[s, pl.ds(0, 16)], pos_v[s, pl.ds(0, 16)])
            plsc.addupdate(rows_v.at[s, pl.ds(16, 16)], pos_v[s, pl.ds(16, 16)])
            return ()

        lax.fori_loop(0, _SEQ, add_body, ())

        pltpu.sync_copy(rows_v, out_hbm.at[pl.ds(base + r * _SEQ, _SEQ)])
        return ()

    lax.fori_loop(0, _ROWS_PER_W, row_body, ())


@functools.partial(
    pl.kernel,
    out_type=jax.ShapeDtypeStruct((_B * _SEQ, _D), jnp.float32),
    mesh=plsc.VectorSubcoreMesh(core_axis_name="c", subcore_axis_name="s"),
    scratch_types=[
        pltpu.VMEM((_IPW,), jnp.int32),
        pltpu.VMEM((_SEQ, _D), jnp.float32),
        pltpu.VMEM((_SEQ, _D), jnp.float32),
        pltpu.SemaphoreType.DMA,
    ],
)
def _embed_sc(ids_hbm, table_hbm, pos_hbm, out_hbm, idx_v, pos_v, rows_v, sem):
    _body(ids_hbm, table_hbm, pos_hbm, out_hbm, idx_v, pos_v, rows_v, sem)


def kernel(token_ids, token_table, pos_table):
    b, seq = token_ids.shape
    ids_flat = token_ids.reshape(b * seq).astype(jnp.int32)
    out = _embed_sc(ids_flat, token_table, pos_table)
    return out.reshape(b, seq, token_table.shape[1])


# speedup vs baseline: 1.2650x; 1.2650x over previous
import functools

import jax
import jax.numpy as jnp
from jax import lax
from jax.experimental import pallas as pl
from jax.experimental.pallas import tpu as pltpu
from jax.experimental.pallas import tpu_sc as plsc

_INFO = plsc.get_sparse_core_info()
_NC, _NS = _INFO.num_cores, _INFO.num_subcores
_NW = _NC * _NS

_B = 4096
_SEQ = 200
_D = 32
_ROWS_PER_W = _B // _NW
_IPW = _ROWS_PER_W * _SEQ


def _body(ids_hbm, table_hbm, pos_hbm, out_hbm, idx_v, pos_v, rows_v, sem):
    wid = lax.axis_index("s") * _NC + lax.axis_index("c")
    base = wid * _IPW

    pltpu.sync_copy(ids_hbm.at[pl.ds(base, _IPW)], idx_v)
    pltpu.sync_copy(pos_hbm.at[pl.ds(0, _SEQ)], pos_v)

    def row_body(r, _):
        pltpu.async_copy(
            table_hbm.at[idx_v.at[pl.ds(r * _SEQ, _SEQ)]], rows_v, sem
        ).wait()

        def add_body(s, _):
            plsc.addupdate(rows_v.at[s, pl.ds(0, 16)], pos_v[s, pl.ds(0, 16)])
            plsc.addupdate(rows_v.at[s, pl.ds(16, 16)], pos_v[s, pl.ds(16, 16)])
            return ()

        lax.fori_loop(0, _SEQ, add_body, ())

        pltpu.sync_copy(rows_v, out_hbm.at[pl.ds(base + r * _SEQ, _SEQ)])
        return ()

    lax.fori_loop(0, _ROWS_PER_W, row_body, ())


@functools.partial(
    pl.kernel,
    out_type=jax.ShapeDtypeStruct((_B * _SEQ, _D), jnp.float32),
    mesh=plsc.VectorSubcoreMesh(core_axis_name="c", subcore_axis_name="s"),
    scratch_types=[
        pltpu.VMEM((_IPW,), jnp.int32),
        pltpu.VMEM((_SEQ, _D), jnp.float32),
        pltpu.VMEM((_SEQ, _D), jnp.float32),
        pltpu.SemaphoreType.DMA,
    ],
    compiler_params=pltpu.CompilerParams(use_tc_tiling_on_sc=False),
)
def _embed_sc(ids_hbm, table_hbm, pos_hbm, out_hbm, idx_v, pos_v, rows_v, sem):
    _body(ids_hbm, table_hbm, pos_hbm, out_hbm, idx_v, pos_v, rows_v, sem)


def kernel(token_ids, token_table, pos_table):
    b, seq = token_ids.shape
    ids_flat = token_ids.reshape(b * seq).astype(jnp.int32)
    out = _embed_sc(ids_flat, token_table, pos_table)
    return out.reshape(b, seq, token_table.shape[1])


# speedup vs baseline: 1.4861x; 1.1748x over previous
import functools

import jax
import jax.numpy as jnp
from jax import lax
from jax.experimental import pallas as pl
from jax.experimental.pallas import tpu as pltpu
from jax.experimental.pallas import tpu_sc as plsc

_INFO = plsc.get_sparse_core_info()
_NC, _NS = _INFO.num_cores, _INFO.num_subcores
_NW = _NC * _NS

_B = 4096
_SEQ = 200
_D = 32
_ROWS_PER_W = _B // _NW
_IPW = _ROWS_PER_W * _SEQ

_C = 2
_CH = _C * _SEQ
_NCHUNK = _ROWS_PER_W // _C
_NBUF = 4
_LOOKAHEAD = 2


def _body(ids_hbm, table_hbm, pos_hbm, out_hbm, idx_v, pos_v, rows, gsems,
          osems):
    wid = lax.axis_index("s") * _NC + lax.axis_index("c")
    base = wid * _IPW

    pltpu.sync_copy(ids_hbm.at[pl.ds(base, _IPW)], idx_v)
    for rep in range(_C):
        pltpu.sync_copy(pos_hbm.at[pl.ds(0, _SEQ)],
                        pos_v.at[pl.ds(rep * _SEQ, _SEQ)])

    def start_gather(c):
        b = c % _NBUF
        return pltpu.async_copy(
            table_hbm.at[idx_v.at[pl.ds(c * _CH, _CH)]], rows[b], gsems[b]
        )

    gather_h = [None] * _NBUF
    out_h = [None] * _NBUF
    for c in range(_LOOKAHEAD):
        gather_h[c % _NBUF] = start_gather(c)

    for c in range(_NCHUNK):
        b = c % _NBUF
        nxt = c + _LOOKAHEAD
        if nxt < _NCHUNK:
            nb = nxt % _NBUF
            if out_h[nb] is not None:
                out_h[nb].wait()
                out_h[nb] = None
            gather_h[nb] = start_gather(nxt)

        gather_h[b].wait()
        rows_b = rows[b]

        @pl.loop(0, _CH, unroll=8)
        def _add(t):
            plsc.addupdate(rows_b.at[t, pl.ds(0, 16)], pos_v[t, pl.ds(0, 16)])
            plsc.addupdate(rows_b.at[t, pl.ds(16, 16)],
                           pos_v[t, pl.ds(16, 16)])

        out_h[b] = pltpu.async_copy(
            rows_b, out_hbm.at[pl.ds(base + c * _CH, _CH)], osems[b])

    for b in range(_NBUF):
        if out_h[b] is not None:
            out_h[b].wait()


@functools.partial(
    pl.kernel,
    out_type=jax.ShapeDtypeStruct((_B * _SEQ, _D), jnp.float32),
    mesh=plsc.VectorSubcoreMesh(core_axis_name="c", subcore_axis_name="s"),
    scratch_types=[
        pltpu.VMEM((_IPW,), jnp.int32),
        pltpu.VMEM((_CH, _D), jnp.float32),
        [pltpu.VMEM((_CH, _D), jnp.float32) for _ in range(_NBUF)],
        [pltpu.SemaphoreType.DMA for _ in range(_NBUF)],
        [pltpu.SemaphoreType.DMA for _ in range(_NBUF)],
    ],
    compiler_params=pltpu.CompilerParams(use_tc_tiling_on_sc=False),
)
def _embed_sc(ids_hbm, table_hbm, pos_hbm, out_hbm, idx_v, pos_v, rows, gsems,
              osems):
    _body(ids_hbm, table_hbm, pos_hbm, out_hbm, idx_v, pos_v, rows, gsems,
          osems)


def kernel(token_ids, token_table, pos_table):
    b, seq = token_ids.shape
    ids_flat = token_ids.reshape(b * seq).astype(jnp.int32)
    out = _embed_sc(ids_flat, token_table, pos_table)
    return out.reshape(b, seq, token_table.shape[1])
